# Initial kernel scaffold; baseline (speedup 1.0000x reference)
import jax
import jax.numpy as jnp
from jax.experimental import pallas as pl


def kernel(x, spa_emb_weight):
    raise NotImplementedError("write your pallas kernel here")


# speedup vs baseline: 1.1038x; 1.1038x over previous
import jax
import jax.numpy as jnp
from jax import lax
from jax.experimental import pallas as pl
from jax.experimental.pallas import tpu as pltpu
from jax.experimental.pallas import tpu_sc as plsc

_B = 16384 * 50
_D = 32
_NC = 2
_NS = 16
_NW = _NC * _NS
_BPW = _B // _NW
_C = 1600
_NCHUNK = _BPW // _C


def _gather_body(idx_hbm, table_hbm, out_hbm, idx_v, rows_v, sem):
    wid = lax.axis_index("s") * _NC + lax.axis_index("c")

    def chunk(i, carry):
        base = wid * _BPW + i * _C
        pltpu.sync_copy(idx_hbm.at[pl.ds(base, _C)], idx_v)
        pltpu.async_copy(table_hbm.at[idx_v], rows_v, sem).wait()
        pltpu.sync_copy(rows_v, out_hbm.at[pl.ds(base, _C)])
        return carry

    lax.fori_loop(0, _NCHUNK, chunk, 0)


@jax.jit
def kernel(x, spa_emb_weight):
    idx = x.astype(jnp.int32).reshape(_B)
    mesh = plsc.VectorSubcoreMesh(core_axis_name="c", subcore_axis_name="s")
    out = pl.kernel(
        _gather_body,
        out_type=jax.ShapeDtypeStruct((_B, _D), jnp.float32),
        mesh=mesh,
        scratch_types=[
            pltpu.VMEM((_C,), jnp.int32),
            pltpu.VMEM((_C, _D), jnp.float32),
            pltpu.SemaphoreType.DMA,
        ],
        compiler_params=pltpu.CompilerParams(use_tc_tiling_on_sc=False),
    )(idx, spa_emb_weight)
    return out.reshape(x.shape[0], x.shape[1], _D)


# speedup vs baseline: 1.1091x; 1.0048x over previous
import jax
import jax.numpy as jnp
from jax import lax
from jax.experimental import pallas as pl
from jax.experimental.pallas import tpu as pltpu
from jax.experimental.pallas import tpu_sc as plsc

_B = 16384 * 50
_D = 32
_NC = 2
_NS = 16
_NW = _NC * _NS
_BPW = _B // _NW
_C = 1600
_NCHUNK = _BPW // _C
_NB = 2


def _gather_body(idx_hbm, table_hbm, out_hbm, idx_v, rows0, rows1,
                 gsem0, gsem1, osem0, osem1):
    wid = lax.axis_index("s") * _NC + lax.axis_index("c")
    base_w = wid * _BPW
    rows = (rows0, rows1)
    gsem = (gsem0, gsem1)
    osem = (osem0, osem1)

    pltpu.sync_copy(idx_hbm.at[wid], idx_v)

    def g_copy(c, b):
        return pltpu.make_async_copy(table_hbm.at[idx_v.at[c]], rows[b], gsem[b])

    def o_copy(c, b):
        return pltpu.make_async_copy(
            rows[b], out_hbm.at[pl.ds(base_w + c * _C, _C)], osem[b])

    g_copy(0, 0).start()
    g_copy(1, 1).start()

    def step(i, carry):
        c0 = 2 * i
        c1 = c0 + 1
        g_copy(c0, 0).wait()
        o_copy(c0, 0).start()
        g_copy(c1, 1).wait()
        o_copy(c1, 1).start()
        o_copy(c0, 0).wait()
        g_copy(c0 + _NB, 0).start()
        o_copy(c1, 1).wait()
        g_copy(c1 + _NB, 1).start()
        return carry

    lax.fori_loop(0, _NCHUNK // 2 - 1, step, 0)

    cl0 = _NCHUNK - 2
    cl1 = _NCHUNK - 1
    g_copy(cl0, 0).wait()
    o_copy(cl0, 0).start()
    g_copy(cl1, 1).wait()
    o_copy(cl1, 1).start()
    o_copy(cl0, 0).wait()
    o_copy(cl1, 1).wait()


@jax.jit
def kernel(x, spa_emb_weight):
    idx = x.astype(jnp.int32).reshape(_NW, _NCHUNK, _C)
    mesh = plsc.VectorSubcoreMesh(core_axis_name="c", subcore_axis_name="s")
    out = pl.kernel(
        _gather_body,
        out_type=jax.ShapeDtypeStruct((_B, _D), jnp.float32),
        mesh=mesh,
        scratch_types=[
            pltpu.VMEM((_NCHUNK, _C), jnp.int32),
            pltpu.VMEM((_C, _D), jnp.float32),
            pltpu.VMEM((_C, _D), jnp.float32),
            pltpu.SemaphoreType.DMA,
            pltpu.SemaphoreType.DMA,
            pltpu.SemaphoreType.DMA,
            pltpu.SemaphoreType.DMA,
        ],
        compiler_params=pltpu.CompilerParams(use_tc_tiling_on_sc=False),
    )(idx, spa_emb_weight)
    return out.reshape(x.shape[0], x.shape[1], _D)


# speedup vs baseline: 1.5148x; 1.3658x over previous
import jax
import jax.numpy as jnp
from jax import lax
from jax.experimental import pallas as pl
from jax.experimental.pallas import tpu as pltpu
from jax.experimental.pallas import tpu_sc as plsc

_NBATCH = 16384
_S = 50
_D = 32
_NC = 2
_NS = 16
_NW = _NC * _NS
_L = 128
_NBHI = _NBATCH // _L
_BHI_PER_W = _NBHI // _NW


def _body(x_hbm, tbl_hbm, p_hbm, xb_v, xbt_v, r0, r1, t0, t1,
          gs0, gs1, os0, os1):
    wid = lax.axis_index("s") * _NC + lax.axis_index("c")
    iota = lax.iota(jnp.int32, 16)
    rbuf = (r0, r1)
    tbuf = (t0, t1)
    gsem = (gs0, gs1)
    osem = (os0, os1)

    def g_copy(s, k):
        return pltpu.make_async_copy(tbl_hbm.at[xbt_v.at[s]], rbuf[k], gsem[k])

    def o_copy(s, bhi, k):
        return pltpu.make_async_copy(tbuf[k], p_hbm.at[s, :, bhi], osem[k])

    def transpose_unit(k):
        for dhi in range(4):
            for dlo in range(8):
                col = jnp.full((16,), dhi * 8 + dlo, jnp.int32)
                for j0 in range(8):
                    rows = iota + (j0 * 16)
                    v = plsc.load_gather(rbuf[k], [rows, col])
                    tbuf[k][dhi, dlo, pl.ds(j0 * 16, 16)] = v

    def do_bhi(bhi, carry):
        b0 = bhi * _L
        pltpu.sync_copy(x_hbm.at[pl.ds(b0, _L)], xb_v)

        def tr_s(s, c):
            cols = jnp.full((16,), s, jnp.int32)
            for j0 in range(8):
                rows = iota + (j0 * 16)
                v = plsc.load_gather(xb_v, [rows, cols])
                xbt_v[s, pl.ds(j0 * 16, 16)] = v
            return c

        lax.fori_loop(0, _S, tr_s, 0)

        g_copy(0, 0).start()

        def step(i, c):
            s0 = 2 * i
            s1 = s0 + 1
            g_copy(s0, 0).wait()
            g_copy(s1, 1).start()

            @pl.when(i > 0)
            def _():
                o_copy(s0 - 2, bhi, 0).wait()

            transpose_unit(0)
            o_copy(s0, bhi, 0).start()
            g_copy(s1, 1).wait()

            @pl.when(s1 < _S - 1)
            def _():
                g_copy(s1 + 1, 0).start()

            @pl.when(i > 0)
            def _():
                o_copy(s1 - 2, bhi, 1).wait()

            transpose_unit(1)
            o_copy(s1, bhi, 1).start()
            return c

        lax.fori_loop(0, _S // 2, step, 0)
        o_copy(_S - 2, bhi, 0).wait()
        o_copy(_S - 1, bhi, 1).wait()
        return carry

    lax.fori_loop(wid * _BHI_PER_W, (wid + 1) * _BHI_PER_W, do_bhi, 0)


@jax.jit
def kernel(x, spa_emb_weight):
    mesh = plsc.VectorSubcoreMesh(core_axis_name="c", subcore_axis_name="s")
    p = pl.kernel(
        _body,
        out_type=jax.ShapeDtypeStruct((_S, _D // 8, _NBHI, 8, _L), jnp.float32),
        mesh=mesh,
        scratch_types=[
            pltpu.VMEM((_L, _S), jnp.int32),
            pltpu.VMEM((_S, _L), jnp.int32),
            pltpu.VMEM((_L, _D), jnp.float32),
            pltpu.VMEM((_L, _D), jnp.float32),
            pltpu.VMEM((_D // 8, 8, _L), jnp.float32),
            pltpu.VMEM((_D // 8, 8, _L), jnp.float32),
            pltpu.SemaphoreType.DMA,
            pltpu.SemaphoreType.DMA,
            pltpu.SemaphoreType.DMA,
            pltpu.SemaphoreType.DMA,
        ],
        compiler_params=pltpu.CompilerParams(
            use_tc_tiling_on_sc=False, needs_layout_passes=False),
    )(x.astype(jnp.int32), spa_emb_weight)
    return p.transpose(2, 4, 0, 1, 3).reshape(_NBATCH, _S, _D)


# speedup vs baseline: 1.6189x; 1.0687x over previous
import jax
import jax.numpy as jnp
from jax import lax
from jax.experimental import pallas as pl
from jax.experimental.pallas import tpu as pltpu
from jax.experimental.pallas import tpu_sc as plsc

_NBATCH = 16384
_S = 50
_D = 32
_NC = 2
_NS = 16
_NW = _NC * _NS
_L = 128
_NBHI = _NBATCH // _L
_BHI_PER_W = _NBHI // _NW
_SC = 5
_NCH = _S // _SC


def _body(x_hbm, tbl_hbm, p_hbm, xb_v, xbt_v, r0, r1, t0, t1,
          gs0, gs1, os0, os1):
    wid = lax.axis_index("s") * _NC + lax.axis_index("c")
    iota = lax.iota(jnp.int32, 16)
    rbuf = (r0, r1)
    tbuf = (t0, t1)
    gsem = (gs0, gs1)
    osem = (os0, os1)

    def g_copy(c, k):
        return pltpu.make_async_copy(
            tbl_hbm.at[xbt_v.at[pl.ds(c * _SC * _L, _SC * _L)]], rbuf[k], gsem[k])

    def o_copy(c, bhi, k):
        return pltpu.make_async_copy(
            tbuf[k], p_hbm.at[pl.ds(c * _SC, _SC), :, bhi], osem[k])

    def transpose_chunk(k):
        def tr(sl, c):
            for dhi in range(4):
                for dlo in range(8):
                    col = jnp.full((16,), dhi * 8 + dlo, jnp.int32)
                    for j0 in range(8):
                        rows = iota + (sl * _L + j0 * 16)
                        v = plsc.load_gather(rbuf[k], [rows, col])
                        tbuf[k][sl, dhi, dlo, pl.ds(j0 * 16, 16)] = v
            return c

        lax.fori_loop(0, _SC, tr, 0)

    def do_bhi(bhi, carry):
        b0 = bhi * _L
        pltpu.sync_copy(x_hbm.at[pl.ds(b0, _L)], xb_v)

        def tr_s(s, c):
            cols = jnp.full((16,), s, jnp.int32)
            for j0 in range(8):
                rows = iota + (j0 * 16)
                v = plsc.load_gather(xb_v, [rows, cols])
                xbt_v[pl.ds(s * _L + j0 * 16, 16)] = v
            return c

        lax.fori_loop(0, _S, tr_s, 0)

        g_copy(0, 0).start()

        def step(i, c):
            c0 = 2 * i
            c1 = c0 + 1
            g_copy(c0, 0).wait()
            g_copy(c1, 1).start()

            @pl.when(i > 0)
            def _():
                o_copy(c0 - 2, bhi, 0).wait()

            transpose_chunk(0)
            o_copy(c0, bhi, 0).start()

            g_copy(c1, 1).wait()

            @pl.when(c1 < _NCH - 1)
            def _():
                g_copy(c1 + 1, 0).start()

            @pl.when(i > 0)
            def _():
                o_copy(c1 - 2, bhi, 1).wait()

            transpose_chunk(1)
            o_copy(c1, bhi, 1).start()
            return c

        lax.fori_loop(0, _NCH // 2, step, 0)
        o_copy(_NCH - 2, bhi, 0).wait()
        o_copy(_NCH - 1, bhi, 1).wait()
        return carry

    lax.fori_loop(wid * _BHI_PER_W, (wid + 1) * _BHI_PER_W, do_bhi, 0)


@jax.jit
def kernel(x, spa_emb_weight):
    mesh = plsc.VectorSubcoreMesh(core_axis_name="c", subcore_axis_name="s")
    p = pl.kernel(
        _body,
        out_type=jax.ShapeDtypeStruct((_S, _D // 8, _NBHI, 8, _L), jnp.float32),
        mesh=mesh,
        scratch_types=[
            pltpu.VMEM((_L, _S), jnp.int32),
            pltpu.VMEM((_S * _L,), jnp.int32),
            pltpu.VMEM((_SC * _L, _D), jnp.float32),
            pltpu.VMEM((_SC * _L, _D), jnp.float32),
            pltpu.VMEM((_SC, _D // 8, 8, _L), jnp.float32),
            pltpu.VMEM((_SC, _D // 8, 8, _L), jnp.float32),
            pltpu.SemaphoreType.DMA,
            pltpu.SemaphoreType.DMA,
            pltpu.SemaphoreType.DMA,
            pltpu.SemaphoreType.DMA,
        ],
        compiler_params=pltpu.CompilerParams(
            use_tc_tiling_on_sc=False, needs_layout_passes=False),
    )(x.astype(jnp.int32), spa_emb_weight)
    return p.transpose(2, 4, 0, 1, 3).reshape(_NBATCH, _S, _D)
